# baseline (speedup 1.0000x reference)
import functools

import jax
import jax.numpy as jnp
from jax import lax
from jax.experimental import pallas as pl
from jax.experimental.pallas import tpu as pltpu
from jax.experimental.pallas import tpu_sc as plsc

_NC = 2
_NS = 16
_NW = _NC * _NS

_GB = 32
_NB = 2
_PD = 128


def _lookup(table_pad, indices, dim):
    batch, fields = indices.shape
    b_per_w = batch // _NW
    n_groups = b_per_w // _GB
    mesh = plsc.VectorSubcoreMesh(core_axis_name="c", subcore_axis_name="s")

    @functools.partial(
        pl.kernel,
        mesh=mesh,
        out_type=jax.ShapeDtypeStruct((batch, fields, dim), jnp.float32),
        scratch_types=[
            pltpu.VMEM((b_per_w, fields), jnp.int32),
            pltpu.VMEM((_NB, _GB, fields, dim), jnp.float32),
            [pltpu.SemaphoreType.DMA] * _NB,
            [pltpu.SemaphoreType.DMA] * _NB,
        ],
        compiler_params=pltpu.CompilerParams(use_tc_tiling_on_sc=False),
    )
    def body(table_hbm, idx_hbm, out_hbm, idx_v, rows_v, gsems, wsems):
        wid = lax.axis_index("s") * _NC + lax.axis_index("c")
        b0 = wid * b_per_w

        def fire_gather(g, b):
            for r in range(_GB):
                pltpu.async_copy(
                    table_hbm.at[idx_v.at[g * _GB + r]],
                    rows_v.at[b, r],
                    gsems[b],
                )

        def wait_gather(g, b):
            for r in range(_GB):
                pltpu.make_async_copy(
                    table_hbm.at[idx_v.at[g * _GB + r]],
                    rows_v.at[b, r],
                    gsems[b],
                ).wait()

        def fire_write(g, b):
            pltpu.async_copy(
                rows_v.at[b],
                out_hbm.at[pl.ds(b0 + g * _GB, _GB)],
                wsems[b],
            )

        def wait_write(g, b):
            pltpu.make_async_copy(
                rows_v.at[b],
                out_hbm.at[pl.ds(b0 + g * _GB, _GB)],
                wsems[b],
            ).wait()

        pltpu.sync_copy(idx_hbm.at[pl.ds(b0, b_per_w)], idx_v)
        for b in range(_NB):
            fire_gather(b, b)

        def step(g2, carry):
            for b in range(_NB):
                g = g2 * _NB + b
                wait_gather(g, b)
                fire_write(g, b)
                wait_write(g, b)
                fire_gather(g + _NB, b)
            return carry

        lax.fori_loop(0, (n_groups - _NB) // _NB, step, 0, unroll=False)

        for b in range(_NB):
            g = n_groups - _NB + b
            wait_gather(g, b)
            fire_write(g, b)
        for b in range(_NB):
            wait_write(n_groups - _NB + b, b)

    return body(table_pad, indices)


_BLKC = 8192


def _widen(table_t):
    dim, num = table_t.shape
    eye = jnp.eye(dim, dtype=jnp.float32)

    def body(e_ref, t_ref, o_ref):
        o_ref[:, 0:dim] = jax.lax.dot_general(
            t_ref[...], e_ref[...], (((0,), (0,)), ((), ())),
            preferred_element_type=jnp.float32,
            precision=jax.lax.Precision.HIGHEST)

    return pl.pallas_call(
        body,
        grid=((num + _BLKC - 1) // _BLKC,),
        in_specs=[pl.BlockSpec((dim, dim), lambda i: (0, 0)),
                  pl.BlockSpec((dim, _BLKC), lambda i: (0, i))],
        out_specs=pl.BlockSpec((_BLKC, _PD), lambda i: (i, 0)),
        out_shape=jax.ShapeDtypeStruct((num, _PD), jnp.float32),
    )(eye, table_t)


def kernel(table, indices):
    num, dim = table.shape
    table_pad = _widen(jnp.transpose(table)).reshape(num * (_PD // dim), dim)
    idx = indices.astype(jnp.int32) * (_PD // dim)
    return _lookup(table_pad, idx, dim)


# speedup vs baseline: 1.2527x; 1.2527x over previous
import functools

import jax
import jax.numpy as jnp
from jax import lax
from jax.experimental import pallas as pl
from jax.experimental.pallas import tpu as pltpu
from jax.experimental.pallas import tpu_sc as plsc

_NC = 2
_NS = 16
_NW = _NC * _NS

_GB = 32
_NB = 2
_PD = 128


def _lookup(table_pad, indices, dim):
    batch, fields = indices.shape
    b_per_w = batch // _NW
    n_groups = b_per_w // _GB
    mesh = plsc.VectorSubcoreMesh(core_axis_name="c", subcore_axis_name="s")

    @functools.partial(
        pl.kernel,
        mesh=mesh,
        out_type=jax.ShapeDtypeStruct((batch, fields, dim), jnp.float32),
        scratch_types=[
            pltpu.VMEM((b_per_w, fields), jnp.int32),
            pltpu.VMEM((_NB, _GB, fields, dim), jnp.float32),
            [pltpu.SemaphoreType.DMA] * _NB,
            [pltpu.SemaphoreType.DMA] * _NB,
        ],
        compiler_params=pltpu.CompilerParams(use_tc_tiling_on_sc=False),
    )
    def body(table_hbm, idx_hbm, out_hbm, idx_v, rows_v, gsems, wsems):
        wid = lax.axis_index("s") * _NC + lax.axis_index("c")
        b0 = wid * b_per_w

        def fire_gather(g, b):
            for r in range(_GB):
                pltpu.async_copy(
                    table_hbm.at[idx_v.at[g * _GB + r]],
                    rows_v.at[b, r],
                    gsems[b],
                )

        def wait_gather(g, b):
            for r in range(_GB):
                pltpu.make_async_copy(
                    table_hbm.at[idx_v.at[g * _GB + r]],
                    rows_v.at[b, r],
                    gsems[b],
                ).wait()

        def fire_write(g, b):
            pltpu.async_copy(
                rows_v.at[b],
                out_hbm.at[pl.ds(b0 + g * _GB, _GB)],
                wsems[b],
            )

        def wait_write(g, b):
            pltpu.make_async_copy(
                rows_v.at[b],
                out_hbm.at[pl.ds(b0 + g * _GB, _GB)],
                wsems[b],
            ).wait()

        pltpu.sync_copy(idx_hbm.at[pl.ds(b0, b_per_w)], idx_v)
        for b in range(_NB):
            fire_gather(b, b)

        def step(g2, carry):
            for b in range(_NB):
                g = g2 * _NB + b
                wait_gather(g, b)
                fire_write(g, b)
                wait_write(g, b)
                fire_gather(g + _NB, b)
            return carry

        lax.fori_loop(0, (n_groups - _NB) // _NB, step, 0, unroll=False)

        for b in range(_NB):
            g = n_groups - _NB + b
            wait_gather(g, b)
            fire_write(g, b)
        for b in range(_NB):
            wait_write(n_groups - _NB + b, b)

    return body(table_pad, indices)


_BLKC = 8192


def _widen(table_t):
    dim, num = table_t.shape
    eye = jnp.eye(dim, dtype=jnp.float32)

    def body(e_ref, t_ref, o_ref):
        o_ref[:, 0:dim] = jax.lax.dot_general(
            t_ref[...], e_ref[...], (((0,), (0,)), ((), ())),
            preferred_element_type=jnp.float32)

    return pl.pallas_call(
        body,
        grid=((num + _BLKC - 1) // _BLKC,),
        in_specs=[pl.BlockSpec((dim, dim), lambda i: (0, 0)),
                  pl.BlockSpec((dim, _BLKC), lambda i: (0, i))],
        out_specs=pl.BlockSpec((_BLKC, _PD), lambda i: (i, 0)),
        out_shape=jax.ShapeDtypeStruct((num, _PD), jnp.float32),
    )(eye, table_t)


def kernel(table, indices):
    num, dim = table.shape
    table_pad = _widen(jnp.transpose(table)).reshape(num * (_PD // dim), dim)
    idx = indices.astype(jnp.int32) * (_PD // dim)
    return _lookup(table_pad, idx, dim)


# speedup vs baseline: 1.2605x; 1.0062x over previous
import functools

import jax
import jax.numpy as jnp
from jax import lax
from jax.experimental import pallas as pl
from jax.experimental.pallas import tpu as pltpu
from jax.experimental.pallas import tpu_sc as plsc

_NC = 2
_NS = 16
_NW = _NC * _NS

_GB = 32
_NB = 2
_PD = 128


def _lookup(table_pad, indices, dim):
    batch, fields = indices.shape
    b_per_w = batch // _NW
    n_groups = b_per_w // _GB
    mesh = plsc.VectorSubcoreMesh(core_axis_name="c", subcore_axis_name="s")

    @functools.partial(
        pl.kernel,
        mesh=mesh,
        out_type=jax.ShapeDtypeStruct((batch, fields, dim), jnp.float32),
        scratch_types=[
            pltpu.VMEM((b_per_w, fields), jnp.int32),
            pltpu.VMEM((_NB, _GB, fields, dim), jnp.float32),
            [pltpu.SemaphoreType.DMA] * _NB,
            [pltpu.SemaphoreType.DMA] * _NB,
        ],
        compiler_params=pltpu.CompilerParams(use_tc_tiling_on_sc=False),
    )
    def body(table_hbm, idx_hbm, out_hbm, idx_v, rows_v, gsems, wsems):
        wid = lax.axis_index("s") * _NC + lax.axis_index("c")
        b0 = wid * b_per_w

        def fire_gather(g, b):
            for r in range(_GB):
                pltpu.async_copy(
                    table_hbm.at[idx_v.at[g * _GB + r]],
                    rows_v.at[b, r],
                    gsems[b],
                )

        def wait_gather(g, b):
            for r in range(_GB):
                pltpu.make_async_copy(
                    table_hbm.at[idx_v.at[g * _GB + r]],
                    rows_v.at[b, r],
                    gsems[b],
                ).wait()

        def fire_write(g, b):
            pltpu.async_copy(
                rows_v.at[b],
                out_hbm.at[pl.ds(b0 + g * _GB, _GB)],
                wsems[b],
            )

        def wait_write(g, b):
            pltpu.make_async_copy(
                rows_v.at[b],
                out_hbm.at[pl.ds(b0 + g * _GB, _GB)],
                wsems[b],
            ).wait()

        pltpu.sync_copy(idx_hbm.at[pl.ds(b0, b_per_w)], idx_v)
        for b in range(_NB):
            fire_gather(b, b)

        def step(g2, carry):
            for b in range(_NB):
                g = g2 * _NB + b
                wait_gather(g, b)
                fire_write(g, b)
                wait_write(g, b)
                fire_gather(g + _NB, b)
            return carry

        lax.fori_loop(0, (n_groups - _NB) // _NB, step, 0, unroll=False)

        for b in range(_NB):
            g = n_groups - _NB + b
            wait_gather(g, b)
            fire_write(g, b)
        for b in range(_NB):
            wait_write(n_groups - _NB + b, b)

    return body(table_pad, indices)


_BLKC = 8192


def _widen(table_t):
    dim, num = table_t.shape
    eye = jnp.eye(dim, dtype=jnp.float32)

    def body(e_ref, t_ref, o_ref):
        o_ref[:, 0:dim] = t_ref[...].T

    return pl.pallas_call(
        body,
        grid=((num + _BLKC - 1) // _BLKC,),
        in_specs=[pl.BlockSpec((dim, dim), lambda i: (0, 0)),
                  pl.BlockSpec((dim, _BLKC), lambda i: (0, i))],
        out_specs=pl.BlockSpec((_BLKC, _PD), lambda i: (i, 0)),
        out_shape=jax.ShapeDtypeStruct((num, _PD), jnp.float32),
    )(eye, table_t)


def kernel(table, indices):
    num, dim = table.shape
    table_pad = _widen(jnp.transpose(table)).reshape(num * (_PD // dim), dim)
    idx = indices.astype(jnp.int32) * (_PD // dim)
    return _lookup(table_pad, idx, dim)


# speedup vs baseline: 1.2998x; 1.0312x over previous
import functools

import jax
import jax.numpy as jnp
from jax import lax
from jax.experimental import pallas as pl
from jax.experimental.pallas import tpu as pltpu
from jax.experimental.pallas import tpu_sc as plsc

_NC = 2
_NS = 16
_NW = _NC * _NS

_GB = 32
_NB = 2
_PD = 128


def _lookup(table_pad, indices, dim):
    batch, fields = indices.shape
    b_per_w = batch // _NW
    n_groups = b_per_w // _GB
    mesh = plsc.VectorSubcoreMesh(core_axis_name="c", subcore_axis_name="s")

    @functools.partial(
        pl.kernel,
        mesh=mesh,
        out_type=jax.ShapeDtypeStruct((batch, fields, dim), jnp.float32),
        scratch_types=[
            pltpu.VMEM((b_per_w, fields), jnp.int32),
            pltpu.VMEM((_NB, _GB, fields, dim), jnp.float32),
            [pltpu.SemaphoreType.DMA] * _NB,
            [pltpu.SemaphoreType.DMA] * _NB,
        ],
        compiler_params=pltpu.CompilerParams(use_tc_tiling_on_sc=False),
    )
    def body(table_hbm, idx_hbm, out_hbm, idx_v, rows_v, gsems, wsems):
        wid = lax.axis_index("s") * _NC + lax.axis_index("c")
        b0 = wid * b_per_w

        def fire_gather(g, b):
            for r in range(_GB):
                pltpu.async_copy(
                    table_hbm.at[idx_v.at[g * _GB + r]],
                    rows_v.at[b, r],
                    gsems[b],
                )

        def wait_gather(g, b):
            for r in range(_GB):
                pltpu.make_async_copy(
                    table_hbm.at[idx_v.at[g * _GB + r]],
                    rows_v.at[b, r],
                    gsems[b],
                ).wait()

        def fire_write(g, b):
            pltpu.async_copy(
                rows_v.at[b],
                out_hbm.at[pl.ds(b0 + g * _GB, _GB)],
                wsems[b],
            )

        def wait_write(g, b):
            pltpu.make_async_copy(
                rows_v.at[b],
                out_hbm.at[pl.ds(b0 + g * _GB, _GB)],
                wsems[b],
            ).wait()

        pltpu.sync_copy(idx_hbm.at[pl.ds(b0, b_per_w)], idx_v)
        for b in range(_NB):
            fire_gather(b, b)

        def step(g2, carry):
            for b in range(_NB):
                g = g2 * _NB + b
                wait_gather(g, b)
                fire_write(g, b)
                wait_write(g, b)
                fire_gather(g + _NB, b)
            return carry

        lax.fori_loop(0, (n_groups - _NB) // _NB, step, 0, unroll=False)

        for b in range(_NB):
            g = n_groups - _NB + b
            wait_gather(g, b)
            fire_write(g, b)
        for b in range(_NB):
            wait_write(n_groups - _NB + b, b)

    return body(table_pad, indices)


_BLKC = 16384


def _widen(table_t):
    dim, num = table_t.shape
    eye = jnp.eye(dim, dtype=jnp.float32)

    def body(e_ref, t_ref, o_ref):
        o_ref[:, 0:dim] = t_ref[...].T

    return pl.pallas_call(
        body,
        grid=((num + _BLKC - 1) // _BLKC,),
        in_specs=[pl.BlockSpec((dim, dim), lambda i: (0, 0)),
                  pl.BlockSpec((dim, _BLKC), lambda i: (0, i))],
        out_specs=pl.BlockSpec((_BLKC, _PD), lambda i: (i, 0)),
        out_shape=jax.ShapeDtypeStruct((num, _PD), jnp.float32),
    )(eye, table_t)


def kernel(table, indices):
    num, dim = table.shape
    table_pad = _widen(jnp.transpose(table)).reshape(num * (_PD // dim), dim)
    idx = indices.astype(jnp.int32) * (_PD // dim)
    return _lookup(table_pad, idx, dim)


# speedup vs baseline: 1.3156x; 1.0121x over previous
import functools

import jax
import jax.numpy as jnp
from jax import lax
from jax.experimental import pallas as pl
from jax.experimental.pallas import tpu as pltpu
from jax.experimental.pallas import tpu_sc as plsc

_NC = 2
_NS = 16
_NW = _NC * _NS

_GB = 32
_NB = 2
_PD = 128


def _lookup(table_pad, indices, dim):
    batch, fields = indices.shape
    b_per_w = batch // _NW
    n_groups = b_per_w // _GB
    mesh = plsc.VectorSubcoreMesh(core_axis_name="c", subcore_axis_name="s")

    @functools.partial(
        pl.kernel,
        mesh=mesh,
        out_type=jax.ShapeDtypeStruct((batch, fields, dim), jnp.float32),
        scratch_types=[
            pltpu.VMEM((b_per_w, fields), jnp.int32),
            pltpu.VMEM((_NB, _GB, fields, dim), jnp.float32),
            [pltpu.SemaphoreType.DMA] * _NB,
            [pltpu.SemaphoreType.DMA] * _NB,
        ],
        compiler_params=pltpu.CompilerParams(use_tc_tiling_on_sc=False),
    )
    def body(table_hbm, idx_hbm, out_hbm, idx_v, rows_v, gsems, wsems):
        wid = lax.axis_index("s") * _NC + lax.axis_index("c")
        b0 = wid * b_per_w

        def fire_gather(g, b):
            for r in range(_GB):
                pltpu.async_copy(
                    table_hbm.at[idx_v.at[g * _GB + r]],
                    rows_v.at[b, r],
                    gsems[b],
                )

        def wait_gather(g, b):
            for r in range(_GB):
                pltpu.make_async_copy(
                    table_hbm.at[idx_v.at[g * _GB + r]],
                    rows_v.at[b, r],
                    gsems[b],
                ).wait()

        def fire_write(g, b):
            pltpu.async_copy(
                rows_v.at[b],
                out_hbm.at[pl.ds(b0 + g * _GB, _GB)],
                wsems[b],
            )

        def wait_write(g, b):
            pltpu.make_async_copy(
                rows_v.at[b],
                out_hbm.at[pl.ds(b0 + g * _GB, _GB)],
                wsems[b],
            ).wait()

        pltpu.sync_copy(idx_hbm.at[pl.ds(b0, b_per_w)], idx_v)
        for b in range(_NB):
            fire_gather(b, b)

        def step(g2, carry):
            for b in range(_NB):
                g = g2 * _NB + b
                wait_gather(g, b)
                fire_write(g, b)
                wait_write(g, b)
                fire_gather(g + _NB, b)
            return carry

        lax.fori_loop(0, (n_groups - _NB) // _NB, step, 0, unroll=False)

        for b in range(_NB):
            g = n_groups - _NB + b
            wait_gather(g, b)
            fire_write(g, b)
        for b in range(_NB):
            wait_write(n_groups - _NB + b, b)

    return body(table_pad, indices)


_BLKC = 32768


def _widen(table_t):
    dim, num = table_t.shape
    eye = jnp.eye(dim, dtype=jnp.float32)

    def body(e_ref, t_ref, o_ref):
        o_ref[:, 0:dim] = t_ref[...].T

    return pl.pallas_call(
        body,
        grid=((num + _BLKC - 1) // _BLKC,),
        in_specs=[pl.BlockSpec((dim, dim), lambda i: (0, 0)),
                  pl.BlockSpec((dim, _BLKC), lambda i: (0, i))],
        out_specs=pl.BlockSpec((_BLKC, _PD), lambda i: (i, 0)),
        out_shape=jax.ShapeDtypeStruct((num, _PD), jnp.float32),
    )(eye, table_t)


def kernel(table, indices):
    num, dim = table.shape
    table_pad = _widen(jnp.transpose(table)).reshape(num * (_PD // dim), dim)
    idx = indices.astype(jnp.int32) * (_PD // dim)
    return _lookup(table_pad, idx, dim)


# speedup vs baseline: 1.3200x; 1.0033x over previous
import functools

import jax
import jax.numpy as jnp
from jax import lax
from jax.experimental import pallas as pl
from jax.experimental.pallas import tpu as pltpu
from jax.experimental.pallas import tpu_sc as plsc

_NC = 2
_NS = 16
_NW = _NC * _NS

_GB = 32
_NB = 2
_PD = 128


def _lookup(table_pad, indices, dim):
    batch, fields = indices.shape
    b_per_w = batch // _NW
    n_groups = b_per_w // _GB
    mesh = plsc.VectorSubcoreMesh(core_axis_name="c", subcore_axis_name="s")

    @functools.partial(
        pl.kernel,
        mesh=mesh,
        out_type=jax.ShapeDtypeStruct((batch, fields, dim), jnp.float32),
        scratch_types=[
            pltpu.VMEM((b_per_w, fields), jnp.int32),
            pltpu.VMEM((_NB, _GB, fields, dim), jnp.float32),
            [pltpu.SemaphoreType.DMA] * _NB,
            [pltpu.SemaphoreType.DMA] * _NB,
        ],
        compiler_params=pltpu.CompilerParams(use_tc_tiling_on_sc=False),
    )
    def body(table_hbm, idx_hbm, out_hbm, idx_v, rows_v, gsems, wsems):
        wid = lax.axis_index("s") * _NC + lax.axis_index("c")
        b0 = wid * b_per_w

        def fire_gather(g, b):
            for r in range(_GB):
                pltpu.async_copy(
                    table_hbm.at[idx_v.at[g * _GB + r]],
                    rows_v.at[b, r],
                    gsems[b],
                )

        def wait_gather(g, b):
            for r in range(_GB):
                pltpu.make_async_copy(
                    table_hbm.at[idx_v.at[g * _GB + r]],
                    rows_v.at[b, r],
                    gsems[b],
                ).wait()

        def fire_write(g, b):
            pltpu.async_copy(
                rows_v.at[b],
                out_hbm.at[pl.ds(b0 + g * _GB, _GB)],
                wsems[b],
            )

        def wait_write(g, b):
            pltpu.make_async_copy(
                rows_v.at[b],
                out_hbm.at[pl.ds(b0 + g * _GB, _GB)],
                wsems[b],
            ).wait()

        pltpu.sync_copy(idx_hbm.at[pl.ds(b0, b_per_w)], idx_v)
        for b in range(_NB):
            fire_gather(b, b)

        def step(g2, carry):
            for b in range(_NB):
                g = g2 * _NB + b
                wait_gather(g, b)
                fire_write(g, b)
                wait_write(g, b)
                fire_gather(g + _NB, b)
            return carry

        lax.fori_loop(0, (n_groups - _NB) // _NB, step, 0, unroll=False)

        for b in range(_NB):
            g = n_groups - _NB + b
            wait_gather(g, b)
            fire_write(g, b)
        for b in range(_NB):
            wait_write(n_groups - _NB + b, b)

    return body(table_pad, indices)


_BLKC = 32768


def _widen(table_t):
    dim, num = table_t.shape

    def body(t_ref, o_ref):
        o_ref[:, 0:dim] = t_ref[...].T

    return pl.pallas_call(
        body,
        grid=((num + _BLKC - 1) // _BLKC,),
        in_specs=[pl.BlockSpec((dim, _BLKC), lambda i: (0, i))],
        out_specs=pl.BlockSpec((_BLKC, _PD), lambda i: (i, 0)),
        out_shape=jax.ShapeDtypeStruct((num, _PD), jnp.float32),
    )(table_t)


def kernel(table, indices):
    num, dim = table.shape
    table_pad = _widen(jnp.transpose(table)).reshape(num * (_PD // dim), dim)
    idx = indices.astype(jnp.int32) * (_PD // dim)
    return _lookup(table_pad, idx, dim)
